# baseline (speedup 1.0000x reference)
import numpy as np
import jax
import jax.numpy as jnp
from jax.experimental import pallas as pl

_NMOL = 64
_MAXA = 32
_NSP = 4
_NRS2 = 24
_NRS3 = 20
_RCUT = 8.0
_ETA2 = 0.32
_ETA3 = 2.7
_TWO_BODY_DECAY = 1.8
_THREE_BODY_DECAY = 0.57
_W3 = float(np.sqrt(_ETA3 / np.pi) * 13.4)
_RS2 = np.linspace(0.0, _RCUT, _NRS2 + 1)[1:].astype(np.float32)
_RS3 = np.linspace(0.0, _RCUT, _NRS3 + 1)[1:].astype(np.float32)
_SPECIES = np.array([1.0, 6.0, 7.0, 8.0], dtype=np.float32)
_PAIRS = [(0, 0), (0, 1), (0, 2), (0, 3), (1, 1), (1, 2), (1, 3),
          (2, 2), (2, 3), (3, 3)]
_NPAIR = len(_PAIRS)
_FP = _NSP * _NRS2 + _NPAIR * _NRS3 * 2

_GM = 4
_NG = _NMOL // _GM
_L = _GM * _MAXA

_DR3 = _RCUT / _NRS3
_C3 = _NRS3 // 2
_RC3 = float(_RS3[_C3])
_UPC = 2.0 * _ETA3 * _DR3
_KUP = [float(np.exp(-_ETA3 * (2 * (s - _C3) + 1) * _DR3 * _DR3))
        for s in range(_C3, _NRS3 - 1)]
_KDN = [float(np.exp(-_ETA3 * (1 - 2 * (s - _C3)) * _DR3 * _DR3))
        for s in range(_C3, 0, -1)]

_FOLDNP = np.zeros((_GM * _NSP * _NSP, _GM * _NPAIR), np.float32)
for _b in range(_GM):
    for _p, (_s, _t) in enumerate(_PAIRS):
        _FOLDNP[_b * 16 + _s * 4 + _t, _b * _NPAIR + _p] = 0.5
        _FOLDNP[_b * 16 + _t * 4 + _s, _b * _NPAIR + _p] = 0.5


def _group_kernel(xi_ref, xr_ref, w_ref, wej_ref, fold_ref, o2_ref, o3_ref):
    A = _MAXA
    L = _L
    W = w_ref[0]
    fold = fold_ref[...]

    sub_i = jax.lax.broadcasted_iota(jnp.int32, (A, L), 0)
    lane_c = jax.lax.broadcasted_iota(jnp.int32, (A, L), 1)
    lane_k = jnp.bitwise_and(lane_c, 31)
    eye = sub_i == lane_k

    d2 = jnp.zeros((A, L), jnp.float32)
    for c in range(3):
        xi = xi_ref[0, c]
        xk = xr_ref[0, c][None, :]
        dx = xi - xk
        d2 = d2 + dx * dx
    d2m = jnp.where(eye, 1.0, d2) + 1e-12
    irr = jax.lax.rsqrt(d2m)
    r = d2m * irr
    in_cut = jnp.logical_and(jnp.logical_not(eye), r < _RCUT)
    fc = jnp.where(in_cut, 0.5 * (jnp.cos((np.pi / _RCUT) * r) + 1.0), 0.0)
    lnr = jnp.log(r)
    pd = jnp.exp(-_THREE_BODY_DECAY * lnr)

    q = 1.0 + _ETA2 * irr * irr
    lnq = jnp.log(q)
    mu = lnr - 0.5 * lnq
    inv_2s2 = 0.5 / lnq
    inv_sig = jax.lax.rsqrt(lnq)
    pref2 = inv_sig * fc * jnp.exp(-_TWO_BODY_DECAY * lnr)
    lnRs2 = np.log(_RS2)
    coef2 = (1.0 / (np.sqrt(2.0 * np.pi) * _RS2)).astype(np.float32)
    y2 = jnp.concatenate(
        [pref2 * float(coef2[s]) *
         jnp.exp(-(float(lnRs2[s]) - mu) ** 2 * inv_2s2)
         for s in range(_NRS2)], axis=0)
    o2_ref[0] = jnp.dot(y2, W, preferred_element_type=jnp.float32, precision=jax.lax.Precision.HIGHEST)

    qs = jnp.concatenate([r, irr, fc, pd], axis=0)
    qsh = qs.astype(jnp.bfloat16)
    qsl = (qs - qsh.astype(jnp.float32)).astype(jnp.bfloat16)
    acc = jnp.zeros((2 * _NRS3 * A, _GM * _NSP * _NSP), jnp.float32)
    for j in range(A):
        mj = jnp.where(
            (jax.lax.broadcasted_iota(jnp.int32, (L, L), 0)
             == jnp.bitwise_and(jax.lax.broadcasted_iota(jnp.int32, (L, L), 1),
                                ~jnp.int32(31)) + j),
            1.0, 0.0).astype(jnp.bfloat16)
        cb = (jnp.dot(qsh, mj, preferred_element_type=jnp.float32)
              + jnp.dot(qsl, mj, preferred_element_type=jnp.float32))
        rij = cb[0 * A:1 * A]
        ir_ij = cb[1 * A:2 * A]
        fc_ij = cb[2 * A:3 * A]
        pd_ij = cb[3 * A:4 * A]
        rjk = qs[0 * A + j][None, :]
        ir_jk = qs[1 * A + j][None, :]
        pd_jk = qs[3 * A + j][None, :]

        rij2 = rij * rij
        rjk2 = rjk * rjk
        rik2 = r * r
        cos_i = jnp.clip((rij2 + rik2 - rjk2) * 0.5 * ir_ij * irr, -1.0, 1.0)
        cos_j = jnp.clip((rij2 + rjk2 - rik2) * 0.5 * ir_ij * ir_jk, -1.0, 1.0)
        cos_k = jnp.clip((rik2 + rjk2 - rij2) * 0.5 * irr * ir_jk, -1.0, 1.0)
        sin_i = jnp.sqrt(jnp.clip(1.0 - cos_i * cos_i, 0.0, 1.0))
        atm = (1.0 + 3.0 * cos_i * cos_j * cos_k) * pd_ij * pd * pd_jk
        neq = jnp.where(lane_k == j, 0.0, 1.0)
        pref = atm * fc_ij * fc * (_W3 * neq)
        rmean = 0.5 * (rij + r)
        a0 = pref * cos_i
        a1 = pref * sin_i

        dm = jnp.minimum(rmean, _RCUT + 0.5) - _RC3
        up = jnp.exp(_UPC * dm)
        dn = 1.0 / up
        rads = [None] * _NRS3
        cur = jnp.exp(-_ETA3 * dm * dm)
        rads[_C3] = cur
        for t, s in enumerate(range(_C3, _NRS3 - 1)):
            cur = cur * up * _KUP[t]
            rads[s + 1] = cur
        cur = rads[_C3]
        for t, s in enumerate(range(_C3, 0, -1)):
            cur = cur * dn * _KDN[t]
            rads[s - 1] = cur
        pieces0 = [a0 * rads[s] for s in range(_NRS3)]
        pieces1 = [a1 * rads[s] for s in range(_NRS3)]
        ys = jnp.concatenate(pieces0 + pieces1, axis=0)
        ysh = ys.astype(jnp.bfloat16)
        ysl = (ys - ysh.astype(jnp.float32)).astype(jnp.bfloat16)
        wej = wej_ref[0, j]
        acc = acc + (jnp.dot(ysh, wej, preferred_element_type=jnp.float32)
                     + jnp.dot(ysl, wej, preferred_element_type=jnp.float32))

    o3_ref[0] = jnp.dot(acc, fold, preferred_element_type=jnp.float32, precision=jax.lax.Precision.HIGHEST)


def kernel(X, Z, atomIDs, molIDs, atom_counts):
    Xg = X.reshape(_NG, _GM, _MAXA, 3)
    XI = jnp.broadcast_to(
        Xg.transpose(0, 3, 2, 1)[:, :, :, :, None],
        (_NG, 3, _MAXA, _GM, _MAXA)).reshape(_NG, 3, _MAXA, _L)
    XR = Xg.transpose(0, 3, 1, 2).reshape(_NG, 3, _L)

    oh = (Z[..., None] == jnp.asarray(_SPECIES)).astype(jnp.float32)
    ohg = oh.reshape(_NG, _GM, _MAXA, _NSP)
    beye = jnp.eye(_GM, dtype=jnp.float32)
    Wm = ohg[:, :, :, None, :] * beye[None, :, None, :, None]
    W = Wm.reshape(_NG, _L, _GM * _NSP)
    teye = jnp.eye(_NSP, dtype=jnp.float32)
    ej = jnp.einsum('bc,tu,gbjs->gjbtcsu', beye, teye, ohg)
    EJ = ej.reshape(_NG, _MAXA, _GM * _NSP, _GM * _NSP * _NSP)
    WEJ = jnp.einsum('glc,gjcd->gjld', W, EJ).astype(jnp.bfloat16)

    fold = jnp.asarray(_FOLDNP)

    from jax.experimental.pallas import tpu as pltpu
    o2, o3 = pl.pallas_call(
        _group_kernel,
        grid=(_NG,),
        compiler_params=pltpu.CompilerParams(
            dimension_semantics=("parallel",)),
        in_specs=[
            pl.BlockSpec((1, 3, _MAXA, _L), lambda g: (g, 0, 0, 0)),
            pl.BlockSpec((1, 3, _L), lambda g: (g, 0, 0)),
            pl.BlockSpec((1, _L, _GM * _NSP), lambda g: (g, 0, 0)),
            pl.BlockSpec((1, _MAXA, _L, _GM * _NSP * _NSP),
                         lambda g: (g, 0, 0, 0)),
            pl.BlockSpec((_GM * _NSP * _NSP, _GM * _NPAIR), lambda g: (0, 0)),
        ],
        out_specs=[
            pl.BlockSpec((1, _NRS2 * _MAXA, _GM * _NSP), lambda g: (g, 0, 0)),
            pl.BlockSpec((1, 2 * _NRS3 * _MAXA, _GM * _NPAIR),
                         lambda g: (g, 0, 0)),
        ],
        out_shape=[
            jax.ShapeDtypeStruct((_NG, _NRS2 * _MAXA, _GM * _NSP), jnp.float32),
            jax.ShapeDtypeStruct((_NG, 2 * _NRS3 * _MAXA, _GM * _NPAIR),
                                 jnp.float32),
        ],
    )(XI, XR, W, WEJ, fold)

    rep2 = o2.reshape(_NG, _NRS2, _MAXA, _GM, _NSP) \
             .transpose(0, 3, 2, 4, 1).reshape(_NMOL, _MAXA, _NSP * _NRS2)
    rep3 = o3.reshape(_NG, 2, _NRS3, _MAXA, _GM, _NPAIR) \
             .transpose(0, 4, 3, 5, 2, 1).reshape(_NMOL, _MAXA,
                                                  _NPAIR * _NRS3 * 2)
    out = jnp.concatenate([rep2, rep3], axis=-1)
    return out.reshape(_NMOL * _MAXA, _FP)


# speedup vs baseline: 1.4082x; 1.4082x over previous
import numpy as np
import jax
import jax.numpy as jnp
from jax.experimental import pallas as pl

_NMOL = 64
_MAXA = 32
_NSP = 4
_NRS2 = 24
_NRS3 = 20
_RCUT = 8.0
_ETA2 = 0.32
_ETA3 = 2.7
_TWO_BODY_DECAY = 1.8
_THREE_BODY_DECAY = 0.57
_W3 = float(np.sqrt(_ETA3 / np.pi) * 13.4)
_RS2 = np.linspace(0.0, _RCUT, _NRS2 + 1)[1:].astype(np.float32)
_RS3 = np.linspace(0.0, _RCUT, _NRS3 + 1)[1:].astype(np.float32)
_SPECIES = np.array([1.0, 6.0, 7.0, 8.0], dtype=np.float32)
_PAIRS = [(0, 0), (0, 1), (0, 2), (0, 3), (1, 1), (1, 2), (1, 3),
          (2, 2), (2, 3), (3, 3)]
_NPAIR = len(_PAIRS)
_FP = _NSP * _NRS2 + _NPAIR * _NRS3 * 2

_GM = 4
_NG = _NMOL // _GM
_L = _GM * _MAXA

_DR3 = _RCUT / _NRS3
_C3 = _NRS3 // 2
_RC3 = float(_RS3[_C3])
_UPC = 2.0 * _ETA3 * _DR3
_KUP = [float(np.exp(-_ETA3 * (2 * (s - _C3) + 1) * _DR3 * _DR3))
        for s in range(_C3, _NRS3 - 1)]
_KDN = [float(np.exp(-_ETA3 * (1 - 2 * (s - _C3)) * _DR3 * _DR3))
        for s in range(_C3, 0, -1)]

_FOLDNP = np.zeros((_GM * _NSP * _NSP, _GM * _NPAIR), np.float32)
for _b in range(_GM):
    for _p, (_s, _t) in enumerate(_PAIRS):
        _FOLDNP[_b * 16 + _s * 4 + _t, _b * _NPAIR + _p] = 0.5
        _FOLDNP[_b * 16 + _t * 4 + _s, _b * _NPAIR + _p] = 0.5


def _group_kernel(xi_ref, xr_ref, w_ref, wej_ref, fold_ref, o2_ref, o3_ref):
    A = _MAXA
    L = _L
    W = w_ref[0]
    fold = fold_ref[...]

    sub_i = jax.lax.broadcasted_iota(jnp.int32, (A, L), 0)
    lane_c = jax.lax.broadcasted_iota(jnp.int32, (A, L), 1)
    lane_k = jnp.bitwise_and(lane_c, 31)
    eye = sub_i == lane_k

    d2 = jnp.zeros((A, L), jnp.float32)
    for c in range(3):
        xi = xi_ref[0, c]
        xk = xr_ref[0, c][None, :]
        dx = xi - xk
        d2 = d2 + dx * dx
    d2m = jnp.where(eye, 1.0, d2) + 1e-12
    irr = jax.lax.rsqrt(d2m)
    r = d2m * irr
    in_cut = jnp.logical_and(jnp.logical_not(eye), r < _RCUT)
    fc = jnp.where(in_cut, 0.5 * (jnp.cos((np.pi / _RCUT) * r) + 1.0), 0.0)
    lnr = jnp.log(r)
    pd = jnp.exp(-_THREE_BODY_DECAY * lnr)

    q = 1.0 + _ETA2 * irr * irr
    lnq = jnp.log(q)
    mu = lnr - 0.5 * lnq
    inv_2s2 = 0.5 / lnq
    inv_sig = jax.lax.rsqrt(lnq)
    pref2 = inv_sig * fc * jnp.exp(-_TWO_BODY_DECAY * lnr)
    lnRs2 = np.log(_RS2)
    coef2 = (1.0 / (np.sqrt(2.0 * np.pi) * _RS2)).astype(np.float32)
    y2 = jnp.concatenate(
        [pref2 * float(coef2[s]) *
         jnp.exp(-(float(lnRs2[s]) - mu) ** 2 * inv_2s2)
         for s in range(_NRS2)], axis=0)
    o2_ref[0] = jnp.dot(y2, W, preferred_element_type=jnp.float32, precision=jax.lax.Precision.HIGHEST)

    qs = jnp.concatenate([r, irr, fc, pd], axis=0)
    qsh = qs.astype(jnp.bfloat16)
    qsl = (qs - qsh.astype(jnp.float32)).astype(jnp.bfloat16)
    acc = jnp.zeros((2 * _NRS3 * A, _GM * _NSP * _NSP), jnp.float32)
    for j in range(A):
        mj = jnp.where(
            (jax.lax.broadcasted_iota(jnp.int32, (L, L), 0)
             == jnp.bitwise_and(jax.lax.broadcasted_iota(jnp.int32, (L, L), 1),
                                ~jnp.int32(31)) + j),
            1.0, 0.0).astype(jnp.bfloat16)
        cb = (jnp.dot(qsh, mj, preferred_element_type=jnp.float32)
              + jnp.dot(qsl, mj, preferred_element_type=jnp.float32))
        rij = cb[0 * A:1 * A]
        ir_ij = cb[1 * A:2 * A]
        fc_ij = cb[2 * A:3 * A]
        pd_ij = cb[3 * A:4 * A]
        rjk = qs[0 * A + j][None, :]
        ir_jk = qs[1 * A + j][None, :]
        pd_jk = qs[3 * A + j][None, :]

        rij2 = rij * rij
        rjk2 = rjk * rjk
        rik2 = r * r
        cos_i = jnp.clip((rij2 + rik2 - rjk2) * 0.5 * ir_ij * irr, -1.0, 1.0)
        cos_j = jnp.clip((rij2 + rjk2 - rik2) * 0.5 * ir_ij * ir_jk, -1.0, 1.0)
        cos_k = jnp.clip((rik2 + rjk2 - rij2) * 0.5 * irr * ir_jk, -1.0, 1.0)
        sin_i = jnp.sqrt(jnp.clip(1.0 - cos_i * cos_i, 0.0, 1.0))
        atm = (1.0 + 3.0 * cos_i * cos_j * cos_k) * pd_ij * pd * pd_jk
        neq = jnp.where(lane_k == j, 0.0, 1.0)
        pref = atm * fc_ij * fc * (_W3 * neq)
        rmean = 0.5 * (rij + r)
        a0 = pref * cos_i
        a1 = pref * sin_i

        dm = jnp.minimum(rmean, _RCUT + 0.5) - _RC3
        up = jnp.exp(_UPC * dm)
        dn = 1.0 / up
        rads = [None] * _NRS3
        cur = jnp.exp(-_ETA3 * dm * dm)
        rads[_C3] = cur
        for t, s in enumerate(range(_C3, _NRS3 - 1)):
            cur = cur * up * _KUP[t]
            rads[s + 1] = cur
        cur = rads[_C3]
        for t, s in enumerate(range(_C3, 0, -1)):
            cur = cur * dn * _KDN[t]
            rads[s - 1] = cur
        pieces0 = [a0 * rads[s] for s in range(_NRS3)]
        pieces1 = [a1 * rads[s] for s in range(_NRS3)]
        ys = jnp.concatenate(pieces0 + pieces1, axis=0)
        ysh = ys.astype(jnp.bfloat16)
        acc = acc + jnp.dot(ysh, wej_ref[0, j],
                            preferred_element_type=jnp.float32)

    o3_ref[0] = jnp.dot(acc, fold, preferred_element_type=jnp.float32, precision=jax.lax.Precision.HIGHEST)


def kernel(X, Z, atomIDs, molIDs, atom_counts):
    Xg = X.reshape(_NG, _GM, _MAXA, 3)
    XI = jnp.broadcast_to(
        Xg.transpose(0, 3, 2, 1)[:, :, :, :, None],
        (_NG, 3, _MAXA, _GM, _MAXA)).reshape(_NG, 3, _MAXA, _L)
    XR = Xg.transpose(0, 3, 1, 2).reshape(_NG, 3, _L)

    oh = (Z[..., None] == jnp.asarray(_SPECIES)).astype(jnp.float32)
    ohg = oh.reshape(_NG, _GM, _MAXA, _NSP)
    beye = jnp.eye(_GM, dtype=jnp.float32)
    Wm = ohg[:, :, :, None, :] * beye[None, :, None, :, None]
    W = Wm.reshape(_NG, _L, _GM * _NSP)
    teye = jnp.eye(_NSP, dtype=jnp.float32)
    ej = jnp.einsum('bc,tu,gbjs->gjbtcsu', beye, teye, ohg)
    EJ = ej.reshape(_NG, _MAXA, _GM * _NSP, _GM * _NSP * _NSP)
    WEJ = jnp.einsum('glc,gjcd->gjld', W, EJ).astype(jnp.bfloat16)

    fold = jnp.asarray(_FOLDNP)

    from jax.experimental.pallas import tpu as pltpu
    o2, o3 = pl.pallas_call(
        _group_kernel,
        grid=(_NG,),
        compiler_params=pltpu.CompilerParams(
            dimension_semantics=("parallel",)),
        in_specs=[
            pl.BlockSpec((1, 3, _MAXA, _L), lambda g: (g, 0, 0, 0)),
            pl.BlockSpec((1, 3, _L), lambda g: (g, 0, 0)),
            pl.BlockSpec((1, _L, _GM * _NSP), lambda g: (g, 0, 0)),
            pl.BlockSpec((1, _MAXA, _L, _GM * _NSP * _NSP),
                         lambda g: (g, 0, 0, 0)),
            pl.BlockSpec((_GM * _NSP * _NSP, _GM * _NPAIR), lambda g: (0, 0)),
        ],
        out_specs=[
            pl.BlockSpec((1, _NRS2 * _MAXA, _GM * _NSP), lambda g: (g, 0, 0)),
            pl.BlockSpec((1, 2 * _NRS3 * _MAXA, _GM * _NPAIR),
                         lambda g: (g, 0, 0)),
        ],
        out_shape=[
            jax.ShapeDtypeStruct((_NG, _NRS2 * _MAXA, _GM * _NSP), jnp.float32),
            jax.ShapeDtypeStruct((_NG, 2 * _NRS3 * _MAXA, _GM * _NPAIR),
                                 jnp.float32),
        ],
    )(XI, XR, W, WEJ, fold)

    rep2 = o2.reshape(_NG, _NRS2, _MAXA, _GM, _NSP) \
             .transpose(0, 3, 2, 4, 1).reshape(_NMOL, _MAXA, _NSP * _NRS2)
    rep3 = o3.reshape(_NG, 2, _NRS3, _MAXA, _GM, _NPAIR) \
             .transpose(0, 4, 3, 5, 2, 1).reshape(_NMOL, _MAXA,
                                                  _NPAIR * _NRS3 * 2)
    out = jnp.concatenate([rep2, rep3], axis=-1)
    return out.reshape(_NMOL * _MAXA, _FP)


# speedup vs baseline: 1.6690x; 1.1851x over previous
import numpy as np
import jax
import jax.numpy as jnp
from jax.experimental import pallas as pl

_NMOL = 64
_MAXA = 32
_NSP = 4
_NRS2 = 24
_NRS3 = 20
_RCUT = 8.0
_ETA2 = 0.32
_ETA3 = 2.7
_TWO_BODY_DECAY = 1.8
_THREE_BODY_DECAY = 0.57
_W3 = float(np.sqrt(_ETA3 / np.pi) * 13.4)
_RS2 = np.linspace(0.0, _RCUT, _NRS2 + 1)[1:].astype(np.float32)
_RS3 = np.linspace(0.0, _RCUT, _NRS3 + 1)[1:].astype(np.float32)
_SPECIES = np.array([1.0, 6.0, 7.0, 8.0], dtype=np.float32)
_PAIRS = [(0, 0), (0, 1), (0, 2), (0, 3), (1, 1), (1, 2), (1, 3),
          (2, 2), (2, 3), (3, 3)]
_NPAIR = len(_PAIRS)
_FP = _NSP * _NRS2 + _NPAIR * _NRS3 * 2

_GM = 4
_NG = _NMOL // _GM
_L = _GM * _MAXA

_DR3 = _RCUT / _NRS3
_C3 = _NRS3 // 2
_RC3 = float(_RS3[_C3])
_UPC = 2.0 * _ETA3 * _DR3
_KUP = [float(np.exp(-_ETA3 * (2 * (s - _C3) + 1) * _DR3 * _DR3))
        for s in range(_C3, _NRS3 - 1)]
_KDN = [float(np.exp(-_ETA3 * (1 - 2 * (s - _C3)) * _DR3 * _DR3))
        for s in range(_C3, 0, -1)]
_KS3 = np.exp(-_ETA3 * ((np.arange(_NRS3) - _C3) * _DR3) ** 2)
_KMATNP = np.broadcast_to(
    np.repeat(np.tile(_KS3, 2), _MAXA).astype(np.float32)[:, None],
    (2 * _NRS3 * _MAXA, _GM * _NPAIR)).copy()
_MJCNP = np.zeros((_L, _MAXA * _L), np.float32)
for _j in range(_MAXA):
    for _b in range(_GM):
        _MJCNP[_b * _MAXA + _j,
               _j * _L + _b * _MAXA:_j * _L + _b * _MAXA + _MAXA] = 1.0

_FOLDNP = np.zeros((_GM * _NSP * _NSP, _GM * _NPAIR), np.float32)
for _b in range(_GM):
    for _p, (_s, _t) in enumerate(_PAIRS):
        _FOLDNP[_b * 16 + _s * 4 + _t, _b * _NPAIR + _p] = 0.5
        _FOLDNP[_b * 16 + _t * 4 + _s, _b * _NPAIR + _p] = 0.5


def _group_kernel(xi_ref, xr_ref, w_ref, wej_ref, mjc_ref, kmat_ref,
                  fold_ref, o2_ref, o3_ref):
    A = _MAXA
    L = _L
    W = w_ref[0]
    fold = fold_ref[...]

    sub_i = jax.lax.broadcasted_iota(jnp.int32, (A, L), 0)
    lane_c = jax.lax.broadcasted_iota(jnp.int32, (A, L), 1)
    lane_k = jnp.bitwise_and(lane_c, 31)
    eye = sub_i == lane_k

    d2 = jnp.zeros((A, L), jnp.float32)
    for c in range(3):
        xi = xi_ref[0, c]
        xk = xr_ref[0, c][None, :]
        dx = xi - xk
        d2 = d2 + dx * dx
    d2m = jnp.where(eye, 1.0, d2) + 1e-12
    irr = jax.lax.rsqrt(d2m)
    r = d2m * irr
    in_cut = jnp.logical_and(jnp.logical_not(eye), r < _RCUT)
    fc = jnp.where(in_cut, 0.5 * (jnp.cos((np.pi / _RCUT) * r) + 1.0), 0.0)
    lnr = jnp.log(r)
    pd = jnp.exp(-_THREE_BODY_DECAY * lnr)

    q = 1.0 + _ETA2 * irr * irr
    lnq = jnp.log(q)
    mu = lnr - 0.5 * lnq
    inv_2s2 = 0.5 / lnq
    inv_sig = jax.lax.rsqrt(lnq)
    pref2 = inv_sig * fc * jnp.exp(-_TWO_BODY_DECAY * lnr)
    lnRs2 = np.log(_RS2)
    coef2 = (1.0 / (np.sqrt(2.0 * np.pi) * _RS2)).astype(np.float32)
    y2 = jnp.concatenate(
        [pref2 * float(coef2[s]) *
         jnp.exp(-(float(lnRs2[s]) - mu) ** 2 * inv_2s2)
         for s in range(_NRS2)], axis=0)
    o2_ref[0] = jnp.dot(y2, W, preferred_element_type=jnp.float32, precision=jax.lax.Precision.HIGHEST)

    qs = jnp.concatenate([r, irr, fc, pd], axis=0)
    qsh = qs.astype(jnp.bfloat16)
    qsl = (qs - qsh.astype(jnp.float32)).astype(jnp.bfloat16)
    mjc = mjc_ref[...]
    CB = (jnp.dot(qsh, mjc, preferred_element_type=jnp.float32)
          + jnp.dot(qsl, mjc, preferred_element_type=jnp.float32))
    acc = jnp.zeros((2 * _NRS3 * A, _GM * _NSP * _NSP), jnp.float32)
    for j in range(A):
        cb = CB[:, j * L:(j + 1) * L]
        rij = cb[0 * A:1 * A]
        ir_ij = cb[1 * A:2 * A]
        fc_ij = cb[2 * A:3 * A]
        pd_ij = cb[3 * A:4 * A]
        rjk = qs[0 * A + j][None, :]
        ir_jk = qs[1 * A + j][None, :]
        pd_jk = qs[3 * A + j][None, :]

        rij2 = rij * rij
        rjk2 = rjk * rjk
        rik2 = r * r
        cos_i = jnp.clip((rij2 + rik2 - rjk2) * 0.5 * ir_ij * irr, -1.0, 1.0)
        cos_j = jnp.clip((rij2 + rjk2 - rik2) * 0.5 * ir_ij * ir_jk, -1.0, 1.0)
        cos_k = jnp.clip((rik2 + rjk2 - rij2) * 0.5 * irr * ir_jk, -1.0, 1.0)
        sin_i = jnp.sqrt(jnp.clip(1.0 - cos_i * cos_i, 0.0, 1.0))
        atm = (1.0 + 3.0 * cos_i * cos_j * cos_k) * pd_ij * pd * pd_jk
        neq = jnp.where(lane_k == j, 0.0, 1.0)
        pref = atm * fc_ij * fc * (_W3 * neq)
        rmean = 0.5 * (rij + r)
        a0 = pref * cos_i
        a1 = pref * sin_i

        dm = jnp.minimum(rmean, _RCUT + 0.5) - _RC3
        up = jnp.exp(_UPC * dm)
        dn = 1.0 / up
        radc = jnp.exp(-_ETA3 * dm * dm)
        pieces0 = [None] * _NRS3
        pieces1 = [None] * _NRS3
        cur0 = a0 * radc
        cur1 = a1 * radc
        pieces0[_C3] = cur0
        pieces1[_C3] = cur1
        for s in range(_C3, _NRS3 - 1):
            cur0 = cur0 * up
            cur1 = cur1 * up
            pieces0[s + 1] = cur0
            pieces1[s + 1] = cur1
        cur0 = pieces0[_C3]
        cur1 = pieces1[_C3]
        for s in range(_C3, 0, -1):
            cur0 = cur0 * dn
            cur1 = cur1 * dn
            pieces0[s - 1] = cur0
            pieces1[s - 1] = cur1
        ys = jnp.concatenate(pieces0 + pieces1, axis=0)
        ysh = ys.astype(jnp.bfloat16)
        acc = acc + jnp.dot(ysh, wej_ref[0, j],
                            preferred_element_type=jnp.float32)

    o3_ref[0] = jnp.dot(acc, fold, preferred_element_type=jnp.float32,
                        precision=jax.lax.Precision.HIGHEST) * kmat_ref[...]


def kernel(X, Z, atomIDs, molIDs, atom_counts):
    Xg = X.reshape(_NG, _GM, _MAXA, 3)
    XI = jnp.broadcast_to(
        Xg.transpose(0, 3, 2, 1)[:, :, :, :, None],
        (_NG, 3, _MAXA, _GM, _MAXA)).reshape(_NG, 3, _MAXA, _L)
    XR = Xg.transpose(0, 3, 1, 2).reshape(_NG, 3, _L)

    oh = (Z[..., None] == jnp.asarray(_SPECIES)).astype(jnp.float32)
    ohg = oh.reshape(_NG, _GM, _MAXA, _NSP)
    beye = jnp.eye(_GM, dtype=jnp.float32)
    Wm = ohg[:, :, :, None, :] * beye[None, :, None, :, None]
    W = Wm.reshape(_NG, _L, _GM * _NSP)
    teye = jnp.eye(_NSP, dtype=jnp.float32)
    ej = jnp.einsum('bc,tu,gbjs->gjbtcsu', beye, teye, ohg)
    EJ = ej.reshape(_NG, _MAXA, _GM * _NSP, _GM * _NSP * _NSP)
    WEJ = jnp.einsum('glc,gjcd->gjld', W, EJ).astype(jnp.bfloat16)

    fold = jnp.asarray(_FOLDNP)

    from jax.experimental.pallas import tpu as pltpu
    o2, o3 = pl.pallas_call(
        _group_kernel,
        grid=(_NG,),
        compiler_params=pltpu.CompilerParams(
            dimension_semantics=("parallel",)),
        in_specs=[
            pl.BlockSpec((1, 3, _MAXA, _L), lambda g: (g, 0, 0, 0)),
            pl.BlockSpec((1, 3, _L), lambda g: (g, 0, 0)),
            pl.BlockSpec((1, _L, _GM * _NSP), lambda g: (g, 0, 0)),
            pl.BlockSpec((1, _MAXA, _L, _GM * _NSP * _NSP),
                         lambda g: (g, 0, 0, 0)),
            pl.BlockSpec((_L, _MAXA * _L), lambda g: (0, 0)),
            pl.BlockSpec((2 * _NRS3 * _MAXA, _GM * _NPAIR), lambda g: (0, 0)),
            pl.BlockSpec((_GM * _NSP * _NSP, _GM * _NPAIR), lambda g: (0, 0)),
        ],
        out_specs=[
            pl.BlockSpec((1, _NRS2 * _MAXA, _GM * _NSP), lambda g: (g, 0, 0)),
            pl.BlockSpec((1, 2 * _NRS3 * _MAXA, _GM * _NPAIR),
                         lambda g: (g, 0, 0)),
        ],
        out_shape=[
            jax.ShapeDtypeStruct((_NG, _NRS2 * _MAXA, _GM * _NSP), jnp.float32),
            jax.ShapeDtypeStruct((_NG, 2 * _NRS3 * _MAXA, _GM * _NPAIR),
                                 jnp.float32),
        ],
    )(XI, XR, W, WEJ, jnp.asarray(_MJCNP).astype(jnp.bfloat16),
      jnp.asarray(_KMATNP), fold)

    rep2 = o2.reshape(_NG, _NRS2, _MAXA, _GM, _NSP) \
             .transpose(0, 3, 2, 4, 1).reshape(_NMOL, _MAXA, _NSP * _NRS2)
    rep3 = o3.reshape(_NG, 2, _NRS3, _MAXA, _GM, _NPAIR) \
             .transpose(0, 4, 3, 5, 2, 1).reshape(_NMOL, _MAXA,
                                                  _NPAIR * _NRS3 * 2)
    out = jnp.concatenate([rep2, rep3], axis=-1)
    return out.reshape(_NMOL * _MAXA, _FP)


# speedup vs baseline: 1.7713x; 1.0613x over previous
import numpy as np
import jax
import jax.numpy as jnp
from jax.experimental import pallas as pl

_NMOL = 64
_MAXA = 32
_NSP = 4
_NRS2 = 24
_NRS3 = 20
_RCUT = 8.0
_ETA2 = 0.32
_ETA3 = 2.7
_TWO_BODY_DECAY = 1.8
_THREE_BODY_DECAY = 0.57
_W3 = float(np.sqrt(_ETA3 / np.pi) * 13.4)
_RS2 = np.linspace(0.0, _RCUT, _NRS2 + 1)[1:].astype(np.float32)
_RS3 = np.linspace(0.0, _RCUT, _NRS3 + 1)[1:].astype(np.float32)
_SPECIES = np.array([1.0, 6.0, 7.0, 8.0], dtype=np.float32)
_PAIRS = [(0, 0), (0, 1), (0, 2), (0, 3), (1, 1), (1, 2), (1, 3),
          (2, 2), (2, 3), (3, 3)]
_NPAIR = len(_PAIRS)
_FP = _NSP * _NRS2 + _NPAIR * _NRS3 * 2

_GM = 4
_NG = _NMOL // _GM
_L = _GM * _MAXA

_DR3 = _RCUT / _NRS3
_C3 = _NRS3 // 2
_RC3 = float(_RS3[_C3])
_UPC = 2.0 * _ETA3 * _DR3
_KUP = [float(np.exp(-_ETA3 * (2 * (s - _C3) + 1) * _DR3 * _DR3))
        for s in range(_C3, _NRS3 - 1)]
_KDN = [float(np.exp(-_ETA3 * (1 - 2 * (s - _C3)) * _DR3 * _DR3))
        for s in range(_C3, 0, -1)]
_KS3 = np.exp(-_ETA3 * ((np.arange(_NRS3) - _C3) * _DR3) ** 2)
_KMATNP = np.broadcast_to(
    np.repeat(np.tile(_KS3, 2), _MAXA).astype(np.float32)[:, None],
    (2 * _NRS3 * _MAXA, _GM * _NPAIR)).copy()
_MJCNP = np.zeros((_L, _MAXA * _L), np.float32)
for _j in range(_MAXA):
    for _b in range(_GM):
        _MJCNP[_b * _MAXA + _j,
               _j * _L + _b * _MAXA:_j * _L + _b * _MAXA + _MAXA] = 1.0

_FOLDNP = np.zeros((_GM * _NSP * _NSP, _GM * _NPAIR), np.float32)
for _b in range(_GM):
    for _p, (_s, _t) in enumerate(_PAIRS):
        _FOLDNP[_b * 16 + _s * 4 + _t, _b * _NPAIR + _p] = 0.5
        _FOLDNP[_b * 16 + _t * 4 + _s, _b * _NPAIR + _p] = 0.5


def _group_kernel(xi_ref, xr_ref, w_ref, wej_ref, mjc_ref, kmat_ref,
                  fold_ref, o2_ref, o3_ref):
    A = _MAXA
    L = _L
    W = w_ref[0]
    fold = fold_ref[...]

    sub_i = jax.lax.broadcasted_iota(jnp.int32, (A, L), 0)
    lane_c = jax.lax.broadcasted_iota(jnp.int32, (A, L), 1)
    lane_k = jnp.bitwise_and(lane_c, 31)
    eye = sub_i == lane_k

    d2 = jnp.zeros((A, L), jnp.float32)
    for c in range(3):
        xi = xi_ref[0, c]
        xk = xr_ref[0, c][None, :]
        dx = xi - xk
        d2 = d2 + dx * dx
    d2m = jnp.where(eye, 1.0, d2) + 1e-12
    irr = jax.lax.rsqrt(d2m)
    r = d2m * irr
    in_cut = jnp.logical_and(jnp.logical_not(eye), r < _RCUT)
    fc = jnp.where(in_cut, 0.5 * (jnp.cos((np.pi / _RCUT) * r) + 1.0), 0.0)
    lnr = jnp.log(r)
    pd = jnp.exp(-_THREE_BODY_DECAY * lnr)

    q = 1.0 + _ETA2 * irr * irr
    lnq = jnp.log(q)
    mu = lnr - 0.5 * lnq
    inv_2s2 = 0.5 / lnq
    inv_sig = jax.lax.rsqrt(lnq)
    pref2 = inv_sig * fc * jnp.exp(-_TWO_BODY_DECAY * lnr)
    lnRs2 = np.log(_RS2)
    coef2 = (1.0 / (np.sqrt(2.0 * np.pi) * _RS2)).astype(np.float32)
    y2 = jnp.concatenate(
        [pref2 * float(coef2[s]) *
         jnp.exp(-(float(lnRs2[s]) - mu) ** 2 * inv_2s2)
         for s in range(_NRS2)], axis=0)
    y2h = y2.astype(jnp.bfloat16)
    y2l = (y2 - y2h.astype(jnp.float32)).astype(jnp.bfloat16)
    Wb = W.astype(jnp.bfloat16)
    o2_ref[0] = (jnp.dot(y2h, Wb, preferred_element_type=jnp.float32)
                 + jnp.dot(y2l, Wb, preferred_element_type=jnp.float32))

    qs = jnp.concatenate([r, irr, fc, pd], axis=0)
    qsh = qs.astype(jnp.bfloat16)
    qsl = (qs - qsh.astype(jnp.float32)).astype(jnp.bfloat16)
    mjc = mjc_ref[...]
    CB = (jnp.dot(qsh, mjc, preferred_element_type=jnp.float32)
          + jnp.dot(qsl, mjc, preferred_element_type=jnp.float32))
    acc = jnp.zeros((2 * _NRS3 * A, _GM * _NSP * _NSP), jnp.float32)
    for j in range(A):
        cb = CB[:, j * L:(j + 1) * L]
        rij = cb[0 * A:1 * A]
        ir_ij = cb[1 * A:2 * A]
        fc_ij = cb[2 * A:3 * A]
        pd_ij = cb[3 * A:4 * A]
        rjk = qs[0 * A + j][None, :]
        ir_jk = qs[1 * A + j][None, :]
        pd_jk = qs[3 * A + j][None, :]

        rij2 = rij * rij
        rjk2 = rjk * rjk
        rik2 = r * r
        cos_i = jnp.clip((rij2 + rik2 - rjk2) * 0.5 * ir_ij * irr, -1.0, 1.0)
        sin_i = jnp.sqrt(jnp.clip(1.0 - cos_i * cos_i, 0.0, 1.0))
        t2 = rij2 + rjk2 - rik2
        t3 = rik2 + rjk2 - rij2
        cjck = (t2 * t3) * ((0.25 * ir_jk * ir_jk) * (ir_ij * irr))
        atm = (1.0 + 3.0 * cos_i * cjck) * (pd_ij * pd * pd_jk)
        neq = jnp.where(lane_k == j, 0.0, 1.0)
        pref = atm * fc_ij * fc * (_W3 * neq)
        rmean = 0.5 * (rij + r)
        a0 = pref * cos_i
        a1 = pref * sin_i

        dm = jnp.minimum(rmean, _RCUT + 0.5) - _RC3
        up = jnp.exp(_UPC * dm)
        dn = 1.0 / up
        radc = jnp.exp(-_ETA3 * dm * dm)
        pieces0 = [None] * _NRS3
        pieces1 = [None] * _NRS3
        cur0 = a0 * radc
        cur1 = a1 * radc
        pieces0[_C3] = cur0
        pieces1[_C3] = cur1
        for s in range(_C3, _NRS3 - 1):
            cur0 = cur0 * up
            cur1 = cur1 * up
            pieces0[s + 1] = cur0
            pieces1[s + 1] = cur1
        cur0 = pieces0[_C3]
        cur1 = pieces1[_C3]
        for s in range(_C3, 0, -1):
            cur0 = cur0 * dn
            cur1 = cur1 * dn
            pieces0[s - 1] = cur0
            pieces1[s - 1] = cur1
        ys = jnp.concatenate(pieces0 + pieces1, axis=0)
        ysh = ys.astype(jnp.bfloat16)
        acc = acc + jnp.dot(ysh, wej_ref[0, j],
                            preferred_element_type=jnp.float32)

    o3_ref[0] = jnp.dot(acc, fold, preferred_element_type=jnp.float32,
                        precision=jax.lax.Precision.HIGHEST) * kmat_ref[...]


def kernel(X, Z, atomIDs, molIDs, atom_counts):
    Xg = X.reshape(_NG, _GM, _MAXA, 3)
    XI = jnp.broadcast_to(
        Xg.transpose(0, 3, 2, 1)[:, :, :, :, None],
        (_NG, 3, _MAXA, _GM, _MAXA)).reshape(_NG, 3, _MAXA, _L)
    XR = Xg.transpose(0, 3, 1, 2).reshape(_NG, 3, _L)

    oh = (Z[..., None] == jnp.asarray(_SPECIES)).astype(jnp.float32)
    ohg = oh.reshape(_NG, _GM, _MAXA, _NSP)
    beye = jnp.eye(_GM, dtype=jnp.float32)
    Wm = ohg[:, :, :, None, :] * beye[None, :, None, :, None]
    W = Wm.reshape(_NG, _L, _GM * _NSP)
    teye = jnp.eye(_NSP, dtype=jnp.float32)
    ej = jnp.einsum('bc,tu,gbjs->gjbtcsu', beye, teye, ohg)
    EJ = ej.reshape(_NG, _MAXA, _GM * _NSP, _GM * _NSP * _NSP)
    WEJ = jnp.einsum('glc,gjcd->gjld', W, EJ).astype(jnp.bfloat16)

    fold = jnp.asarray(_FOLDNP)

    from jax.experimental.pallas import tpu as pltpu
    o2, o3 = pl.pallas_call(
        _group_kernel,
        grid=(_NG,),
        compiler_params=pltpu.CompilerParams(
            dimension_semantics=("parallel",)),
        in_specs=[
            pl.BlockSpec((1, 3, _MAXA, _L), lambda g: (g, 0, 0, 0)),
            pl.BlockSpec((1, 3, _L), lambda g: (g, 0, 0)),
            pl.BlockSpec((1, _L, _GM * _NSP), lambda g: (g, 0, 0)),
            pl.BlockSpec((1, _MAXA, _L, _GM * _NSP * _NSP),
                         lambda g: (g, 0, 0, 0)),
            pl.BlockSpec((_L, _MAXA * _L), lambda g: (0, 0)),
            pl.BlockSpec((2 * _NRS3 * _MAXA, _GM * _NPAIR), lambda g: (0, 0)),
            pl.BlockSpec((_GM * _NSP * _NSP, _GM * _NPAIR), lambda g: (0, 0)),
        ],
        out_specs=[
            pl.BlockSpec((1, _NRS2 * _MAXA, _GM * _NSP), lambda g: (g, 0, 0)),
            pl.BlockSpec((1, 2 * _NRS3 * _MAXA, _GM * _NPAIR),
                         lambda g: (g, 0, 0)),
        ],
        out_shape=[
            jax.ShapeDtypeStruct((_NG, _NRS2 * _MAXA, _GM * _NSP), jnp.float32),
            jax.ShapeDtypeStruct((_NG, 2 * _NRS3 * _MAXA, _GM * _NPAIR),
                                 jnp.float32),
        ],
    )(XI, XR, W, WEJ, jnp.asarray(_MJCNP).astype(jnp.bfloat16),
      jnp.asarray(_KMATNP), fold)

    rep2 = o2.reshape(_NG, _NRS2, _MAXA, _GM, _NSP) \
             .transpose(0, 3, 2, 4, 1).reshape(_NMOL, _MAXA, _NSP * _NRS2)
    rep3 = o3.reshape(_NG, 2, _NRS3, _MAXA, _GM, _NPAIR) \
             .transpose(0, 4, 3, 5, 2, 1).reshape(_NMOL, _MAXA,
                                                  _NPAIR * _NRS3 * 2)
    out = jnp.concatenate([rep2, rep3], axis=-1)
    return out.reshape(_NMOL * _MAXA, _FP)
